# Initial kernel scaffold; baseline (speedup 1.0000x reference)
import jax
import jax.numpy as jnp
from jax.experimental import pallas as pl


def kernel(pred, target):
    raise NotImplementedError("write your pallas kernel here")


# speedup vs baseline: 13.7614x; 13.7614x over previous
import functools

import jax
import jax.numpy as jnp
from jax import lax
from jax.experimental import pallas as pl
from jax.experimental.pallas import tpu as pltpu
from jax.experimental.pallas import tpu_sc as plsc

S = 16
N = 512 * 512
NB = 2048
NBF = float(NB)
NC, NS, L = 2, 16, 16
PER_W = N // 2
CHUNK = 8192
NCHUNK = PER_W // CHUNK
SAMPLES_PER_CORE = S // NC


def _sc_body(pred_hbm, targ_hbm, out_hbm, pbuf, tbuf, hist, bufa, bufb,
             lossbuf, shared, sem):
    c = lax.axis_index("c")
    s = lax.axis_index("s")
    sample = c * SAMPLES_PER_CORE + s // 2
    half = s % 2
    base = sample * N + half * PER_W

    def _zero(i, _):
        hist[pl.ds(i * L, L)] = jnp.zeros((L,), jnp.float32)
        return 0
    lax.fori_loop(0, 2 * NB // L, _zero, 0)

    ones = jnp.full((L,), 1.0, jnp.float32)

    def _chunk(k, _):
        off = base + k * CHUNK
        pltpu.sync_copy(pred_hbm.at[pl.ds(off, CHUNK)], pbuf)
        pltpu.sync_copy(targ_hbm.at[pl.ds(off, CHUNK)], tbuf)

        def _vec(i, _):
            p = pbuf[pl.ds(i * L, L)]
            t = tbuf[pl.ds(i * L, L)]
            z = p * (1.0 - 2.0 * t)
            e = 1.0 / (1.0 + jnp.exp(-z))
            bi = jnp.minimum(
                (e * NBF).astype(jnp.int32), NB - 1
            ) + t.astype(jnp.int32) * NB
            plsc.addupdate_scatter(hist, [bi], ones)
            return 0
        lax.fori_loop(0, CHUNK // L, _vec, 0)
        return 0
    lax.fori_loop(0, NCHUNK, _chunk, 0)

    pltpu.sync_copy(hist, shared.at[s])
    plsc.subcore_barrier()

    @pl.when(s < SAMPLES_PER_CORE)
    def _phase2():
        pltpu.sync_copy(shared.at[2 * s], bufa)
        pltpu.sync_copy(shared.at[2 * s + 1], bufb)

        def _tot(i, acc):
            off = NB + i * L
            return acc + bufa[pl.ds(off, L)] + bufb[pl.ds(off, L)]
        pos_tot = lax.fori_loop(0, NB // L, _tot, jnp.zeros((L,), jnp.float32))
        p_total = jnp.sum(pos_tot)

        rev_iota = (15 - lax.iota(jnp.int32, L)).astype(jnp.float32)

        def _scan(k, carry):
            cp0, cn0, lacc = carry
            b0 = NB - (k + 1) * L
            neg = bufa[pl.ds(b0, L)] + bufb[pl.ds(b0, L)]
            pos = bufa[pl.ds(NB + b0, L)] + bufb[pl.ds(NB + b0, L)]
            nrev = lax.rev(neg, (0,))
            prev_ = lax.rev(pos, (0,))
            cp_inc = cp0 + plsc.cumsum(prev_)
            cn_inc = cn0 + plsc.cumsum(nrev)
            cp_exc = cp_inc - prev_
            cn_exc = cn_inc - nrev
            j_inc = (cp_inc + cn_inc) / jnp.maximum(p_total + cn_inc, 1.0)
            j_exc = (cp_exc + cn_exc) / jnp.maximum(p_total + cn_exc, 1.0)
            center = (jnp.float32(b0) + rev_iota + 0.5) * (1.0 / NBF)
            lacc = lacc + center * (j_inc - j_exc)
            cp1 = cp0 + jnp.sum(pos)
            cn1 = cn0 + jnp.sum(neg)
            return cp1, cn1, lacc
        _, _, lacc = lax.fori_loop(
            0, NB // L, _scan,
            (jnp.float32(0.0), jnp.float32(0.0), jnp.zeros((L,), jnp.float32)))

        loss = jnp.sum(lacc)
        lossbuf[...] = jnp.full((L,), loss, jnp.float32)
        out_sample = c * SAMPLES_PER_CORE + s
        pltpu.sync_copy(lossbuf, out_hbm.at[out_sample])


@functools.partial(
    pl.kernel,
    out_type=jax.ShapeDtypeStruct((S, L), jnp.float32),
    mesh=plsc.VectorSubcoreMesh(core_axis_name="c", subcore_axis_name="s"),
    scratch_types=[
        pltpu.VMEM((CHUNK,), jnp.float32),
        pltpu.VMEM((CHUNK,), jnp.float32),
        pltpu.VMEM((2 * NB,), jnp.float32),
        pltpu.VMEM((2 * NB,), jnp.float32),
        pltpu.VMEM((2 * NB,), jnp.float32),
        pltpu.VMEM((L,), jnp.float32),
        pltpu.VMEM_SHARED((NS, 2 * NB), jnp.float32),
        pltpu.SemaphoreType.DMA,
    ],
    compiler_params=pltpu.CompilerParams(needs_layout_passes=False),
)
def _sc_hist_loss(pred_hbm, targ_hbm, out_hbm, *scratch):
    _sc_body(pred_hbm, targ_hbm, out_hbm, *scratch)


def _mean_body(x_ref, o_ref):
    o_ref[...] = jnp.sum(x_ref[:, 0:1], keepdims=True) * (1.0 / S)


_mean_call = pl.pallas_call(
    _mean_body,
    out_shape=jax.ShapeDtypeStruct((1, 1), jnp.float32),
)


def kernel(pred, target):
    pred_flat = pred.reshape(S * N)
    targ_flat = target.reshape(S * N)
    losses = _sc_hist_loss(pred_flat, targ_flat)
    return _mean_call(losses).reshape(())


# speedup vs baseline: 45.3034x; 3.2921x over previous
import functools

import jax
import jax.numpy as jnp
from jax import lax
from jax.experimental import pallas as pl
from jax.experimental.pallas import tpu as pltpu
from jax.experimental.pallas import tpu_sc as plsc

S = 16
N = 512 * 512
NB = 2048
NBF = float(NB)
NC, NS, L = 2, 16, 16
PER_W = N // 2
CHUNK = 16384
NCHUNK = PER_W // CHUNK
SAMPLES_PER_CORE = S // NC

_TCR = 128


def _bins_body(p_ref, t_ref, o_ref):
    p = p_ref[0, 0]
    t = t_ref[0, 0]
    z2 = p * (t + t - 1.0)
    e = 1.0 / (1.0 + jnp.exp(z2))
    o_ref[0, 0] = (e * 2047.99 + t * 2048.0).astype(jnp.int32)


_bins_call = pl.pallas_call(
    _bins_body,
    grid=(S, 512 // _TCR),
    in_specs=[
        pl.BlockSpec((1, 1, _TCR, 512), lambda s, g: (s, 0, g, 0)),
        pl.BlockSpec((1, 1, _TCR, 512), lambda s, g: (s, 0, g, 0)),
    ],
    out_specs=pl.BlockSpec((1, 1, _TCR, 512), lambda s, g: (s, 0, g, 0)),
    out_shape=jax.ShapeDtypeStruct((S, 1, 512, 512), jnp.int32),
)


def _sc_body(bins_hbm, out_hbm, bbuf0, bbuf1, hist, bufa, bufb,
             lossbuf, shared, sem0, sem1):
    c = lax.axis_index("c")
    s = lax.axis_index("s")
    sample = c * SAMPLES_PER_CORE + s // 2
    half = s % 2
    base = sample * N + half * PER_W

    def _zero(i, _):
        hist[pl.ds(i * L, L)] = jnp.zeros((L,), jnp.float32)
        return 0
    lax.fori_loop(0, 2 * NB // L, _zero, 0)

    ones = jnp.full((L,), 1.0, jnp.float32)
    U = 16

    bufs = ((bbuf0, sem0), (bbuf1, sem1))

    def _start(k, b):
        off = base + k * CHUNK
        bb, sm = bufs[b]
        pltpu.async_copy(bins_hbm.at[pl.ds(off, CHUNK)], bb, sm)

    def _wait(k, b):
        off = base + k * CHUNK
        bb, sm = bufs[b]
        pltpu.make_async_copy(bins_hbm.at[pl.ds(off, CHUNK)], bb, sm).wait()

    def _process(b):
        bb, _ = bufs[b]

        def _vec(i, _):
            bis = []
            for u in range(U):
                bis.append(bb[pl.ds((i * U + u) * L, L)])
            for bi in bis:
                plsc.addupdate_scatter(hist, [bi], ones)
            return 0
        lax.fori_loop(0, CHUNK // (L * U), _vec, 0)

    _start(0, 0)
    for k in range(NCHUNK):
        b = k % 2
        if k + 1 < NCHUNK:
            _start(k + 1, 1 - b)
        _wait(k, b)
        _process(b)

    pltpu.sync_copy(hist, shared.at[s])
    plsc.subcore_barrier()

    @pl.when(s < SAMPLES_PER_CORE)
    def _phase2():
        pltpu.sync_copy(shared.at[2 * s], bufa)
        pltpu.sync_copy(shared.at[2 * s + 1], bufb)

        def _tot(i, acc):
            r = acc
            for u in range(4):
                off = NB + (i * 4 + u) * L
                r = r + bufa[pl.ds(off, L)] + bufb[pl.ds(off, L)]
            return r
        pos_tot = lax.fori_loop(0, NB // L // 4, _tot,
                                jnp.zeros((L,), jnp.float32))
        p_total = jnp.sum(pos_tot)

        rev_iota = (15 - lax.iota(jnp.int32, L)).astype(jnp.float32)

        def _scan(k, carry):
            cp0, cn0, lacc = carry
            b0 = NB - (k + 1) * L
            neg = bufa[pl.ds(b0, L)] + bufb[pl.ds(b0, L)]
            pos = bufa[pl.ds(NB + b0, L)] + bufb[pl.ds(NB + b0, L)]
            nrev = lax.rev(neg, (0,))
            prev_ = lax.rev(pos, (0,))
            cp_inc = cp0 + plsc.cumsum(prev_)
            cn_inc = cn0 + plsc.cumsum(nrev)
            cp_exc = cp_inc - prev_
            cn_exc = cn_inc - nrev
            j_inc = (cp_inc + cn_inc) / jnp.maximum(p_total + cn_inc, 1.0)
            j_exc = (cp_exc + cn_exc) / jnp.maximum(p_total + cn_exc, 1.0)
            center = (jnp.float32(b0) + rev_iota + 0.5) * (1.0 / NBF)
            lacc = lacc + center * (j_inc - j_exc)
            cp1 = cp0 + jnp.sum(pos)
            cn1 = cn0 + jnp.sum(neg)
            return cp1, cn1, lacc
        _, _, lacc = lax.fori_loop(
            0, NB // L, _scan,
            (jnp.float32(0.0), jnp.float32(0.0), jnp.zeros((L,), jnp.float32)))

        loss = jnp.sum(lacc)
        lossbuf[...] = jnp.full((L,), loss, jnp.float32)
        out_sample = c * SAMPLES_PER_CORE + s
        pltpu.sync_copy(lossbuf, out_hbm.at[out_sample])


@functools.partial(
    pl.kernel,
    out_type=jax.ShapeDtypeStruct((S, L), jnp.float32),
    mesh=plsc.VectorSubcoreMesh(core_axis_name="c", subcore_axis_name="s"),
    scratch_types=[
        pltpu.VMEM((CHUNK,), jnp.int32),
        pltpu.VMEM((CHUNK,), jnp.int32),
        pltpu.VMEM((2 * NB,), jnp.float32),
        pltpu.VMEM((2 * NB,), jnp.float32),
        pltpu.VMEM((2 * NB,), jnp.float32),
        pltpu.VMEM((L,), jnp.float32),
        pltpu.VMEM_SHARED((NS, 2 * NB), jnp.float32),
        pltpu.SemaphoreType.DMA,
        pltpu.SemaphoreType.DMA,
    ],
    compiler_params=pltpu.CompilerParams(needs_layout_passes=False),
)
def _sc_hist_loss(bins_hbm, out_hbm, *scratch):
    _sc_body(bins_hbm, out_hbm, *scratch)


def _mean_body(x_ref, o_ref):
    o_ref[...] = jnp.sum(x_ref[:, 0:1], keepdims=True) * (1.0 / S)


_mean_call = pl.pallas_call(
    _mean_body,
    out_shape=jax.ShapeDtypeStruct((1, 1), jnp.float32),
)


def kernel(pred, target):
    bins = _bins_call(pred, target).reshape(S * N)
    losses = _sc_hist_loss(bins)
    return _mean_call(losses).reshape(())


# speedup vs baseline: 74.8803x; 1.6529x over previous
import functools

import jax
import jax.numpy as jnp
from jax import lax
from jax.experimental import pallas as pl
from jax.experimental.pallas import tpu as pltpu
from jax.experimental.pallas import tpu_sc as plsc

S = 16
N = 512 * 512
W = 512
NB = 2048
NBF = float(NB)
NC, NS, L = 2, 16, 16
ROWS_W = 256
CROWS = 32
CHUNK = CROWS * W
NCHUNK = ROWS_W // CROWS
SAMPLES_PER_CORE = S // NC


def _sc_body(pred_hbm, targ_hbm, out_hbm, pbuf0, tbuf0, pbuf1, tbuf1,
             hist, bufa, bufb, lossbuf, shared, sem0, sem1):
    c = lax.axis_index("c")
    s = lax.axis_index("s")
    sample = c * SAMPLES_PER_CORE + s // 2
    half = s % 2
    base_row = half * ROWS_W

    def _zero(i, _):
        hist[pl.ds(i * L, L)] = jnp.zeros((L,), jnp.float32)
        return 0
    lax.fori_loop(0, 2 * NB // L, _zero, 0)

    ones = jnp.full((L,), 1.0, jnp.float32)
    U = 16

    bufs = ((pbuf0, tbuf0, sem0), (pbuf1, tbuf1, sem1))

    def _start(k, b):
        r0 = base_row + k * CROWS
        pb, tb, sm = bufs[b]
        pltpu.async_copy(pred_hbm.at[sample, 0, pl.ds(r0, CROWS), :], pb, sm)
        pltpu.async_copy(targ_hbm.at[sample, 0, pl.ds(r0, CROWS), :], tb, sm)

    def _wait(k, b):
        r0 = base_row + k * CROWS
        pb, tb, sm = bufs[b]
        pltpu.make_async_copy(
            pred_hbm.at[sample, 0, pl.ds(r0, CROWS), :], pb, sm).wait()
        pltpu.make_async_copy(
            targ_hbm.at[sample, 0, pl.ds(r0, CROWS), :], tb, sm).wait()

    def _process(b):
        pb, tb, _ = bufs[b]

        def _vec(i, _):
            a = i // 2
            c0 = (i % 2) * (W // 2)
            bis = []
            for u in range(U):
                p = pb[a, pl.ds(c0 + u * L, L)]
                t = tb[a, pl.ds(c0 + u * L, L)]
                z2 = p * (t + t - 1.0)
                e = 1.0 / (1.0 + jnp.exp(z2))
                bis.append((e * 2047.99 + t * 2048.0).astype(jnp.int32))
            for bi in bis:
                plsc.addupdate_scatter(hist, [bi], ones)
            return 0
        lax.fori_loop(0, CHUNK // (L * U), _vec, 0)

    _start(0, 0)
    for k in range(NCHUNK):
        b = k % 2
        if k + 1 < NCHUNK:
            _start(k + 1, 1 - b)
        _wait(k, b)
        _process(b)

    pltpu.sync_copy(hist, shared.at[s])
    plsc.subcore_barrier()

    @pl.when(s < SAMPLES_PER_CORE)
    def _phase2():
        pltpu.sync_copy(shared.at[2 * s], bufa)
        pltpu.sync_copy(shared.at[2 * s + 1], bufb)

        def _tot(i, acc):
            r = acc
            for u in range(4):
                off = NB + (i * 4 + u) * L
                r = r + bufa[pl.ds(off, L)] + bufb[pl.ds(off, L)]
            return r
        pos_tot = lax.fori_loop(0, NB // L // 4, _tot,
                                jnp.zeros((L,), jnp.float32))
        p_total = jnp.sum(pos_tot)

        rev_iota = (15 - lax.iota(jnp.int32, L)).astype(jnp.float32)

        def _scan(k, carry):
            cp0, cn0, lacc = carry
            b0 = NB - (k + 1) * L
            neg = bufa[pl.ds(b0, L)] + bufb[pl.ds(b0, L)]
            pos = bufa[pl.ds(NB + b0, L)] + bufb[pl.ds(NB + b0, L)]
            nrev = lax.rev(neg, (0,))
            prev_ = lax.rev(pos, (0,))
            cp_inc = cp0 + plsc.cumsum(prev_)
            cn_inc = cn0 + plsc.cumsum(nrev)
            cp_exc = cp_inc - prev_
            cn_exc = cn_inc - nrev
            j_inc = (cp_inc + cn_inc) / jnp.maximum(p_total + cn_inc, 1.0)
            j_exc = (cp_exc + cn_exc) / jnp.maximum(p_total + cn_exc, 1.0)
            center = (jnp.float32(b0) + rev_iota + 0.5) * (1.0 / NBF)
            lacc = lacc + center * (j_inc - j_exc)
            cp1 = cp0 + jnp.sum(pos)
            cn1 = cn0 + jnp.sum(neg)
            return cp1, cn1, lacc
        _, _, lacc = lax.fori_loop(
            0, NB // L, _scan,
            (jnp.float32(0.0), jnp.float32(0.0), jnp.zeros((L,), jnp.float32)))

        loss = jnp.sum(lacc)
        lossbuf[...] = jnp.full((L,), loss, jnp.float32)
        out_sample = c * SAMPLES_PER_CORE + s
        pltpu.sync_copy(lossbuf, out_hbm.at[out_sample])


@functools.partial(
    pl.kernel,
    out_type=jax.ShapeDtypeStruct((S, L), jnp.float32),
    mesh=plsc.VectorSubcoreMesh(core_axis_name="c", subcore_axis_name="s"),
    scratch_types=[
        pltpu.VMEM((CROWS, W), jnp.float32),
        pltpu.VMEM((CROWS, W), jnp.float32),
        pltpu.VMEM((CROWS, W), jnp.float32),
        pltpu.VMEM((CROWS, W), jnp.float32),
        pltpu.VMEM((2 * NB,), jnp.float32),
        pltpu.VMEM((2 * NB,), jnp.float32),
        pltpu.VMEM((2 * NB,), jnp.float32),
        pltpu.VMEM((L,), jnp.float32),
        pltpu.VMEM_SHARED((NS, 2 * NB), jnp.float32),
        pltpu.SemaphoreType.DMA,
        pltpu.SemaphoreType.DMA,
    ],
    compiler_params=pltpu.CompilerParams(needs_layout_passes=False),
)
def _sc_hist_loss(pred_hbm, targ_hbm, out_hbm, *scratch):
    _sc_body(pred_hbm, targ_hbm, out_hbm, *scratch)


def _mean_body(x_ref, o_ref):
    o_ref[...] = jnp.sum(x_ref[:, 0:1], keepdims=True) * (1.0 / S)


_mean_call = pl.pallas_call(
    _mean_body,
    out_shape=jax.ShapeDtypeStruct((1, 1), jnp.float32),
)


def kernel(pred, target):
    losses = _sc_hist_loss(pred, target)
    return _mean_call(losses).reshape(())
